# baseline (speedup 1.0000x reference)
import functools

import jax
import jax.numpy as jnp
from jax import lax
from jax.experimental import pallas as pl
from jax.experimental.pallas import tpu as pltpu
from jax.experimental.pallas import tpu_sc as plsc

B = 16384
H = 64

_info = plsc.get_sparse_core_info()
_NC = _info.num_cores
_NS = _info.num_subcores
_NW = _NC * _NS
_BPW = B // _NW
_CH = 128
_NCH = _BPW // _CH

_mesh = plsc.VectorSubcoreMesh(core_axis_name="c", subcore_axis_name="s")


@functools.partial(
    pl.kernel,
    mesh=_mesh,
    compiler_params=pltpu.CompilerParams(use_tc_tiling_on_sc=False),
    out_type=(
        jax.ShapeDtypeStruct((B, H), jnp.float32),
        jax.ShapeDtypeStruct((B, H), jnp.float32),
    ),
    scratch_types=[
        pltpu.VMEM((_BPW,), jnp.int32),
        pltpu.VMEM((_BPW,), jnp.int32),
        pltpu.VMEM((_BPW, H), jnp.float32),
        pltpu.VMEM((_BPW, H), jnp.float32),
        pltpu.SemaphoreType.DMA,
    ],
)
def _gather_kernel(kgg_ids, rel_ids, kgg_emb, rel_emb, kgg_out, rel_out,
                   kidx, ridx, krows, rrows, sem):
    wid = lax.axis_index("s") * _NC + lax.axis_index("c")
    base = wid * _BPW
    pltpu.sync_copy(kgg_ids.at[pl.ds(base, _BPW)], kidx)
    pltpu.sync_copy(rel_ids.at[pl.ds(base, _BPW)], ridx)
    copies = []
    for c in range(_NCH):
        sl = pl.ds(c * _CH, _CH)
        copies.append(pltpu.async_copy(kgg_emb.at[kidx.at[sl]], krows.at[sl], sem))
        copies.append(pltpu.async_copy(rel_emb.at[ridx.at[sl]], rrows.at[sl], sem))
    for cp in copies:
        cp.wait()
    pltpu.sync_copy(krows, kgg_out.at[pl.ds(base, _BPW)])
    pltpu.sync_copy(rrows, rel_out.at[pl.ds(base, _BPW)])


def kernel(kgg_ids, relation_ids, kgg_embedding, relation_embedding):
    kgg_out, rel_out = _gather_kernel(
        kgg_ids.astype(jnp.int32), relation_ids.astype(jnp.int32),
        kgg_embedding, relation_embedding)
    return (kgg_out, rel_out)
